# Initial kernel scaffold; baseline (speedup 1.0000x reference)
import jax
import jax.numpy as jnp
from jax.experimental import pallas as pl


def kernel(tokens, byte_embed, hash_tables, conv_w, conv_b, in_proj_w, in_proj_b, w1, w2, wo, ln_g, ln_b, head_w, head_b):
    raise NotImplementedError("write your pallas kernel here")


# speedup vs baseline: 3.1481x; 3.1481x over previous
import functools

import jax
import jax.numpy as jnp
from jax import lax
from jax.experimental import pallas as pl
from jax.experimental.pallas import tpu as pltpu
from jax.experimental.pallas import tpu_sc as plsc

_HASH_PRIMES = [2654435761, 2246822519, 3266489917, 2028178513, 1220703125,
                1610612741, 805306457, 402653189, 3674653429, 2860486313,
                1073676287, 2971215073, 1500450271, 3267000013, 2654435789,
                4049292737, 2246822531, 3266489927, 2028178519, 1220703133]
_FIB = [1, 1, 2, 3, 5, 8, 13, 21]
_MATCH_OFFSETS = (1, 2, 3, 4, 5, 6, 7, 8, 12, 16, 24, 32)
_B, _T = 32, 512
_VOCAB, _BYTE_DIM = 1024, 128
_NUM_TABLES, _BUCKETS, _EPT = 8, 32768, 16
_HIDDEN, _NUM_LAYERS, _KSZ = 512, 2, 8
_PAD = 32


def _sc_gather(tokens_pad, ht_flat, byte_embed):
    mesh = plsc.VectorSubcoreMesh(core_axis_name="c", subcore_axis_name="s")
    n_chunks = _T // 128

    @functools.partial(
        pl.kernel,
        out_type=(
            jax.ShapeDtypeStruct((_B, _T, _BYTE_DIM), jnp.float32),
            jax.ShapeDtypeStruct((_NUM_TABLES, _B, _T, _EPT), jnp.float32),
        ),
        mesh=mesh,
        compiler_params=pltpu.CompilerParams(use_tc_tiling_on_sc=False),
        scratch_types=[
            pltpu.VMEM((_PAD + _T,), jnp.int32),
            pltpu.VMEM((_NUM_TABLES * n_chunks, 128), jnp.int32),
            pltpu.VMEM((128, _EPT), jnp.float32),
            pltpu.VMEM((128, _BYTE_DIM), jnp.float32),
            pltpu.SemaphoreType.DMA,
        ],
    )
    def k(tokp_hbm, ht_hbm, be_hbm, byte_hbm, hf_hbm, tok_v, idx_v, hbuf, bbuf, sem):
        wid = lax.axis_index("s") * 2 + lax.axis_index("c")
        b = wid

        pltpu.sync_copy(tokp_hbm.at[b], tok_v)

        for i in range(_NUM_TABLES):
            off = _FIB[i]
            prime15 = _HASH_PRIMES[(i * 3) % len(_HASH_PRIMES)] & 32767
            for j in range(n_chunks):
                for g in range(128 // 16):
                    t0 = j * 128 + g * 16
                    tok16 = tok_v[pl.ds(_PAD - off + t0, 16)]
                    idx16 = ((tok16 * prime15) & 32767) + i * _BUCKETS
                    idx_v[i * n_chunks + j, pl.ds(g * 16, 16)] = idx16

        for i in range(_NUM_TABLES):
            for j in range(n_chunks):
                pltpu.async_copy(
                    ht_hbm.at[idx_v.at[jnp.int32(i * n_chunks + j)]], hbuf, sem).wait()
                pltpu.sync_copy(
                    hbuf,
                    hf_hbm.at[jnp.int32(i), b, pl.ds(j * 128, 128)])

        for j in range(n_chunks):
            pltpu.async_copy(
                be_hbm.at[tok_v.at[pl.ds(_PAD + j * 128, 128)]], bbuf, sem).wait()
            pltpu.sync_copy(bbuf, byte_hbm.at[b, pl.ds(j * 128, 128)])

    return k(tokens_pad, ht_flat, byte_embed)


def _trunk_body(tok_ref, byte_ref, hf_ref, cw_ref, cb_ref, wb_ref, wh_ref,
                wm_ref, ipb_ref, w1_ref, w2_ref, wo_ref, lng_ref, lnb_ref,
                hw_ref, hb_ref, out_ref):
    bf = byte_ref[0]
    hf = jnp.concatenate([hf_ref[i, 0] for i in range(_NUM_TABLES)], axis=1)
    tcol = tok_ref[0]

    acc = hf * cw_ref[7][None, :]
    for s in range(1, _KSZ):
        shifted = jnp.concatenate(
            [jnp.zeros((s, 128), jnp.float32), hf[:_T - s]], axis=0)
        acc = acc + shifted * cw_ref[7 - s][None, :]
    hconv = acc + cb_ref[0][None, :]

    row = lax.broadcasted_iota(jnp.int32, (_T, 1), 0)
    cols = []
    for off in _MATCH_OFFSETS:
        shifted = jnp.concatenate(
            [jnp.zeros((off, 1), jnp.float32), tcol[:_T - off]], axis=0)
        eq = ((tcol == shifted) & (row >= off)).astype(jnp.float32)
        cols.append(eq)
    cols.append(jnp.zeros((_T, 4), jnp.float32))
    match = jnp.concatenate(cols, axis=1)

    f32 = jnp.float32
    h = (jnp.dot(bf, wb_ref[...], preferred_element_type=f32)
         + jnp.dot(hconv, wh_ref[...], preferred_element_type=f32)
         + jnp.dot(match, wm_ref[...], preferred_element_type=f32)
         + ipb_ref[0][None, :])

    for l in range(_NUM_LAYERS):
        u = jnp.dot(h, w1_ref[l], preferred_element_type=f32)
        v = jnp.dot(h, w2_ref[l], preferred_element_type=f32)
        a = (u * jax.nn.sigmoid(u)) * v
        a = jnp.dot(a, wo_ref[l], preferred_element_type=f32)
        x = a + h
        m = jnp.mean(x, axis=-1, keepdims=True)
        xc = x - m
        var = jnp.mean(xc * xc, axis=-1, keepdims=True)
        h = xc * lax.rsqrt(var + 1e-5) * lng_ref[l][None, :] + lnb_ref[l][None, :]

    out_ref[0] = (jnp.dot(h, hw_ref[...], preferred_element_type=f32)
                  + hb_ref[0][None, :])


def _trunk(tok_col, byte_feat, hfeat, cw8, cb, wb, wh, wm, ipb, w1t, w2t, wot,
           ln_g, ln_b, hwt, hb, interpret=False):
    _z = lambda: jnp.int32(0)
    full = lambda *shape: pl.BlockSpec(
        shape, lambda b, _n=len(shape): tuple(_z() for _ in range(_n)))
    per_b = lambda *shape: pl.BlockSpec(
        (1,) + shape, lambda b, _n=len(shape): (b,) + tuple(_z() for _ in range(_n)))
    return pl.pallas_call(
        _trunk_body,
        grid=(_B,),
        in_specs=[
            per_b(_T, 1),
            per_b(_T, _BYTE_DIM),
            pl.BlockSpec((_NUM_TABLES, 1, _T, _EPT),
                         lambda b: (jnp.int32(0), b, jnp.int32(0),
                                    jnp.int32(0))),
            full(_KSZ, 128),
            full(1, 128),
            full(128, _HIDDEN),
            full(128, _HIDDEN),
            full(16, _HIDDEN),
            full(1, _HIDDEN),
            full(_NUM_LAYERS, _HIDDEN, _HIDDEN),
            full(_NUM_LAYERS, _HIDDEN, _HIDDEN),
            full(_NUM_LAYERS, _HIDDEN, _HIDDEN),
            full(_NUM_LAYERS, _HIDDEN),
            full(_NUM_LAYERS, _HIDDEN),
            full(_HIDDEN, _VOCAB),
            full(1, _VOCAB),
        ],
        out_specs=per_b(_T, _VOCAB),
        out_shape=jax.ShapeDtypeStruct((_B, _T, _VOCAB), jnp.float32),
        interpret=interpret,
    )(tok_col, byte_feat, hfeat, cw8, cb, wb, wh, wm, ipb, w1t, w2t, wot,
      ln_g, ln_b, hwt, hb)


def kernel(tokens, byte_embed, hash_tables, conv_w, conv_b, in_proj_w,
           in_proj_b, w1, w2, wo, ln_g, ln_b, head_w, head_b):
    tokens_i32 = tokens.astype(jnp.int32)
    tokens_pad = jnp.pad(tokens_i32, ((0, 0), (_PAD, 0)))
    ht_flat = hash_tables.reshape(_NUM_TABLES * _BUCKETS, _EPT)

    _TMP_JNP_GATHER = False
    if _TMP_JNP_GATHER:
        byte_feat = byte_embed[tokens_i32]
        hfs = []
        for i in range(_NUM_TABLES):
            off = _FIB[i]
            prime15 = _HASH_PRIMES[(i * 3) % 20] & 32767
            shifted = tokens_pad[:, _PAD - off:_PAD - off + _T]
            idx = ((shifted * prime15) & 32767) + i * _BUCKETS
            hfs.append(ht_flat[idx])
        hfeat = jnp.stack(hfs, axis=0)
    else:
        byte_feat, hfeat = _sc_gather(tokens_pad, ht_flat, byte_embed)

    tok_col = tokens_i32.astype(jnp.float32)[:, :, None]
    cw8 = conv_w[:, 0, :].T
    wb = in_proj_w[:, :128].T
    wh = in_proj_w[:, 128:256].T
    wm = jnp.pad(in_proj_w[:, 256:268].T, ((0, 4), (0, 0)))
    w1t = jnp.transpose(w1, (0, 2, 1))
    w2t = jnp.transpose(w2, (0, 2, 1))
    wot = jnp.transpose(wo, (0, 2, 1))
    return _trunk(tok_col, byte_feat, hfeat, cw8, conv_b[None, :], wb, wh, wm,
                  in_proj_b[None, :], w1t, w2t, wot, ln_g, ln_b, head_w.T,
                  head_b[None, :])
